# Initial kernel scaffold; baseline (speedup 1.0000x reference)
import jax
import jax.numpy as jnp
from jax.experimental import pallas as pl


def kernel(audio_semantic, video_semantic, embedding, ema_count, ema_weight):
    raise NotImplementedError("write your pallas kernel here")


# speedup vs baseline: 2.2935x; 2.2935x over previous
import jax
import jax.numpy as jnp
import numpy as np
from jax.experimental import pallas as pl

B, T, D, M = 128, 64, 256, 1024
N = B * T
R = 512
RB = R // T
DECAY, EPS = 0.99, 1e-05
MAXENT = np.log(M)


def _encode_kernel(x_ref, y_ref, emb_ref,
                   q_ref, ph1_ref, ph05_ref, wsum_ref, csum_ref, amax_ref):
    i = pl.program_id(0)
    x = x_ref[...]
    y = y_ref[...]
    emb = emb_ref[...]

    e2 = jnp.sum(emb * emb, axis=1, keepdims=True).T
    x2 = jnp.sum(x * x, axis=1, keepdims=True)
    xe = jax.lax.dot_general(x, emb, (((1,), (1,)), ((), ())))
    d = e2 + x2 - 2.0 * xe

    iota_m = jax.lax.broadcasted_iota(jnp.int32, (R, M), 1)
    dmin = jnp.min(d, axis=1, keepdims=True)
    idx = jnp.min(jnp.where(d == dmin, iota_m, M), axis=1, keepdims=True)
    onehot = (iota_m == idx).astype(jnp.float32)

    sq = jnp.sqrt(jnp.maximum(d, 0.0))
    z = -sq
    z1 = z - jnp.max(z, axis=1, keepdims=True)
    ez1 = jnp.exp(z1)
    ph1 = ez1 / jnp.sum(ez1, axis=1, keepdims=True)
    z2 = z * 2.0
    z2 = z2 - jnp.max(z2, axis=1, keepdims=True)
    ez2 = jnp.exp(z2)
    ph05 = ez2 / jnp.sum(ez2, axis=1, keepdims=True)

    ent = -jnp.sum(ph1 * jnp.log(ph1 + 1e-5), axis=1, keepdims=True)
    adj = 1.0 - ent / MAXENT

    q = jax.lax.dot_general(onehot, emb, (((1,), (0,)), ((), ())))
    q_ref[...] = x + (q - x)
    ph1_ref[...] = jnp.transpose(ph1.reshape(RB, T, M), (1, 0, 2))
    ph05_ref[...] = jnp.transpose(ph05.reshape(RB, T, M), (1, 0, 2))

    enc2 = adj * onehot
    w = jax.lax.dot_general(enc2, x + y, (((0,), (0,)), ((), ())))
    c = jnp.sum(enc2, axis=0, keepdims=True)

    @pl.when(i == 0)
    def _init():
        wsum_ref[...] = w
        csum_ref[...] = c

    @pl.when(i > 0)
    def _acc():
        wsum_ref[...] += w
        csum_ref[...] += c

    counts = jnp.sum(onehot.reshape(RB, T, M), axis=1)
    cmax = jnp.max(counts, axis=1, keepdims=True)
    iota_b = jax.lax.broadcasted_iota(jnp.int32, (RB, M), 1)
    amax_ref[...] = jnp.min(jnp.where(counts == cmax, iota_b, M),
                            axis=1, keepdims=True)


def _scode_kernel(a05_ref, v1_ref, v05_ref, a1_ref, s1_ref, s2_ref):
    a05 = a05_ref[0, :, :]
    l1 = jnp.log(v1_ref[0, :, :] + 1e-10)
    s1_ref[0, :, :] = jax.lax.dot_general(
        a05, l1, (((1,), (1,)), ((), ())))
    v05 = v05_ref[0, :, :]
    l2 = jnp.log(a1_ref[0, :, :] + 1e-10)
    s2_ref[0, :, :] = jax.lax.dot_general(
        v05, l2, (((1,), (1,)), ((), ())))


def _lcmcm_from_scode(S):
    mx = jnp.max(-S)
    es = jnp.exp(S + mx)
    sums = jnp.sum(es, axis=-1)
    eye = (jax.lax.broadcasted_iota(jnp.int32, (B, B), 0)
           == jax.lax.broadcasted_iota(jnp.int32, (B, B), 1)).astype(jnp.float32)
    diag = jnp.sum(es * eye[None, :, :], axis=-1)
    return -jnp.mean(jnp.log(diag / (sums + EPS)))


def _final_kernel(s1_ref, s2_ref, csa_ref, csv_ref, wsa_ref, wsv_ref,
                  cnt_ref, wgt_ref, ama_ref, amv_ref,
                  cm_ref, eq_ref, emb_out_ref):
    l1 = _lcmcm_from_scode(s1_ref[...])
    l2 = _lcmcm_from_scode(s2_ref[...])
    cm_ref[...] = jnp.reshape(0.5 * (l1 + l2), (1, 1))

    csa = csa_ref[...]
    csv = csv_ref[...]
    ec = DECAY * cnt_ref[...] + (1.0 - DECAY) * csv
    n = jnp.sum(ec)
    ec = (ec + EPS) / (n + M * EPS) * n
    ew = DECAY * wgt_ref[...] + 0.5 * (1.0 - DECAY) * wsv_ref[...]
    ec2 = DECAY * ec + (1.0 - DECAY) * csa
    n2 = jnp.sum(ec2)
    ec2 = (ec2 + EPS) / (n2 + M * EPS) * n2
    ew2 = DECAY * ew + 0.5 * (1.0 - DECAY) * wsa_ref[...]
    emb_out_ref[...] = ew2 / ec2.T

    eq_ref[...] = jnp.reshape(
        jnp.sum((ama_ref[...] == amv_ref[...]).astype(jnp.int32)), (1, 1))


def kernel(audio_semantic, video_semantic, embedding, ema_count, ema_weight):
    a_flat = audio_semantic.reshape(N, D)
    v_flat = video_semantic.reshape(N, D)

    enc = pl.pallas_call(
        _encode_kernel,
        grid=(N // R,),
        in_specs=[
            pl.BlockSpec((R, D), lambda i: (i, 0)),
            pl.BlockSpec((R, D), lambda i: (i, 0)),
            pl.BlockSpec((M, D), lambda i: (0, 0)),
        ],
        out_specs=[
            pl.BlockSpec((R, D), lambda i: (i, 0)),
            pl.BlockSpec((T, RB, M), lambda i: (0, i, 0)),
            pl.BlockSpec((T, RB, M), lambda i: (0, i, 0)),
            pl.BlockSpec((M, D), lambda i: (0, 0)),
            pl.BlockSpec((1, M), lambda i: (0, 0)),
            pl.BlockSpec((RB, 1), lambda i: (i, 0)),
        ],
        out_shape=[
            jax.ShapeDtypeStruct((N, D), jnp.float32),
            jax.ShapeDtypeStruct((T, B, M), jnp.float32),
            jax.ShapeDtypeStruct((T, B, M), jnp.float32),
            jax.ShapeDtypeStruct((M, D), jnp.float32),
            jax.ShapeDtypeStruct((1, M), jnp.float32),
            jax.ShapeDtypeStruct((B, 1), jnp.int32),
        ],
    )
    a_q, a_ph1, a_ph05, wsum_a, csum_a, amax_a = enc(a_flat, v_flat, embedding)
    v_q, v_ph1, v_ph05, wsum_v, csum_v, amax_v = enc(v_flat, a_flat, embedding)

    scode = pl.pallas_call(
        _scode_kernel,
        grid=(T,),
        in_specs=[pl.BlockSpec((1, B, M), lambda t: (t, 0, 0))] * 4,
        out_specs=[pl.BlockSpec((1, B, B), lambda t: (t, 0, 0))] * 2,
        out_shape=[jax.ShapeDtypeStruct((T, B, B), jnp.float32)] * 2,
    )
    s1, s2 = scode(a_ph05, v_ph1, v_ph05, a_ph1)

    cm, eq, new_embedding = pl.pallas_call(
        _final_kernel,
        out_shape=[
            jax.ShapeDtypeStruct((1, 1), jnp.float32),
            jax.ShapeDtypeStruct((1, 1), jnp.int32),
            jax.ShapeDtypeStruct((M, D), jnp.float32),
        ],
    )(s1, s2, csum_a, csum_v, wsum_a, wsum_v,
      ema_count.reshape(1, M), ema_weight, amax_a, amax_v)

    return (a_q.reshape(B, T, D), v_q.reshape(B, T, D),
            cm[0, 0], eq[0, 0], new_embedding)


# speedup vs baseline: 4.1505x; 1.8096x over previous
import jax
import jax.numpy as jnp
import numpy as np
from jax.experimental import pallas as pl
from jax.experimental.pallas import tpu as pltpu

B, T, D, M = 128, 64, 256, 1024
TB = 8
NT = TB * B
NBLK = T // TB
DECAY, EPS = 0.99, 1e-05
MAXENT = np.log(M)


def _lcmcm_from_scode(S):
    mx = jnp.max(-S)
    es = jnp.exp(S + mx)
    sums = jnp.sum(es, axis=-1)
    eye = (jax.lax.broadcasted_iota(jnp.int32, (B, B), 0)
           == jax.lax.broadcasted_iota(jnp.int32, (B, B), 1)).astype(jnp.float32)
    diag = jnp.sum(es * eye[None, :, :], axis=-1)
    return -jnp.mean(jnp.log(diag / (sums + EPS)))


def _fused_kernel(a_ref, v_ref, emb_ref, cnt_ref, wgt_ref,
                  aq_ref, vq_ref, cm_ref, eq_ref, emb_out_ref,
                  s1_ref, s2_ref, wsa_ref, wsv_ref,
                  csa_ref, csv_ref, ca_ref, cv_ref):
    i = pl.program_id(0)
    emb = emb_ref[...]
    e2 = jnp.sum(emb * emb, axis=1, keepdims=True).T

    def encode(x, y):
        x2 = jnp.sum(x * x, axis=1, keepdims=True)
        xe = jax.lax.dot_general(x, emb, (((1,), (1,)), ((), ())))
        d = e2 + x2 - 2.0 * xe

        iota_m = jax.lax.broadcasted_iota(jnp.int32, (NT, M), 1)
        dmin = jnp.min(d, axis=1, keepdims=True)
        onehot = (jnp.min(jnp.where(d == dmin, iota_m, M), axis=1, keepdims=True)
                  == iota_m).astype(jnp.float32)

        z = -jnp.sqrt(jnp.maximum(d, 0.0))
        z1 = z - jnp.max(z, axis=1, keepdims=True)
        ez1 = jnp.exp(z1)
        ph1 = ez1 / jnp.sum(ez1, axis=1, keepdims=True)
        z2 = 2.0 * z
        z2 = z2 - jnp.max(z2, axis=1, keepdims=True)
        ez2 = jnp.exp(z2)
        ph05 = ez2 / jnp.sum(ez2, axis=1, keepdims=True)

        ent = -jnp.sum(ph1 * jnp.log(ph1 + 1e-5), axis=1, keepdims=True)
        adj = 1.0 - ent / MAXENT

        q = jax.lax.dot_general(onehot, emb, (((1,), (0,)), ((), ())))
        enc2 = adj * onehot
        w = jax.lax.dot_general(enc2, x + y, (((0,), (0,)), ((), ())))
        c = jnp.sum(enc2, axis=0, keepdims=True)
        return q, ph1, ph05, onehot, w, c

    x = jnp.swapaxes(a_ref[...], 0, 1).reshape(NT, D)
    y = jnp.swapaxes(v_ref[...], 0, 1).reshape(NT, D)
    qa, ph1a, ph05a, oha, wa, csa = encode(x, y)
    qv, ph1v, ph05v, ohv, wv, csv = encode(y, x)
    aq_ref[...] = jnp.swapaxes(qa.reshape(TB, B, D), 0, 1)
    vq_ref[...] = jnp.swapaxes(qv.reshape(TB, B, D), 0, 1)

    la = jnp.log(ph1a + 1e-10)
    lv = jnp.log(ph1v + 1e-10)
    s1 = []
    s2 = []
    for tt in range(TB):
        sl = slice(tt * B, (tt + 1) * B)
        s1.append(jax.lax.dot_general(
            ph05a[sl], lv[sl], (((1,), (1,)), ((), ()))))
        s2.append(jax.lax.dot_general(
            ph05v[sl], la[sl], (((1,), (1,)), ((), ()))))
    s1_ref[pl.ds(i * TB, TB)] = jnp.stack(s1, axis=0)
    s2_ref[pl.ds(i * TB, TB)] = jnp.stack(s2, axis=0)

    cnt_a = jnp.sum(oha.reshape(TB, B, M), axis=0)
    cnt_v = jnp.sum(ohv.reshape(TB, B, M), axis=0)

    @pl.when(i == 0)
    def _init():
        wsa_ref[...] = wa
        wsv_ref[...] = wv
        csa_ref[...] = csa
        csv_ref[...] = csv
        ca_ref[...] = cnt_a
        cv_ref[...] = cnt_v

    @pl.when(i > 0)
    def _acc():
        wsa_ref[...] += wa
        wsv_ref[...] += wv
        csa_ref[...] += csa
        csv_ref[...] += csv
        ca_ref[...] += cnt_a
        cv_ref[...] += cnt_v

    @pl.when(i == NBLK - 1)
    def _finale():
        l1 = _lcmcm_from_scode(s1_ref[...])
        l2 = _lcmcm_from_scode(s2_ref[...])
        cm_ref[...] = jnp.reshape(0.5 * (l1 + l2), (1, 1))

        counts_a = ca_ref[...]
        counts_v = cv_ref[...]
        iota_b = jax.lax.broadcasted_iota(jnp.int32, (B, M), 1)
        ama = jnp.min(jnp.where(counts_a == jnp.max(counts_a, axis=1, keepdims=True),
                                iota_b, M), axis=1, keepdims=True)
        amv = jnp.min(jnp.where(counts_v == jnp.max(counts_v, axis=1, keepdims=True),
                                iota_b, M), axis=1, keepdims=True)
        eq_ref[...] = jnp.reshape(jnp.sum((ama == amv).astype(jnp.int32)), (1, 1))

        ec = DECAY * cnt_ref[...] + (1.0 - DECAY) * csv_ref[...]
        n = jnp.sum(ec)
        ec = (ec + EPS) / (n + M * EPS) * n
        ew = DECAY * wgt_ref[...] + 0.5 * (1.0 - DECAY) * wsv_ref[...]
        ec2 = DECAY * ec + (1.0 - DECAY) * csa_ref[...]
        n2 = jnp.sum(ec2)
        ec2 = (ec2 + EPS) / (n2 + M * EPS) * n2
        ew2 = DECAY * ew + 0.5 * (1.0 - DECAY) * wsa_ref[...]
        emb_out_ref[...] = ew2 / ec2.T


def kernel(audio_semantic, video_semantic, embedding, ema_count, ema_weight):
    a_q, v_q, cm, eq, new_embedding = pl.pallas_call(
        _fused_kernel,
        grid=(NBLK,),
        in_specs=[
            pl.BlockSpec((B, TB, D), lambda t: (0, t, 0)),
            pl.BlockSpec((B, TB, D), lambda t: (0, t, 0)),
            pl.BlockSpec((M, D), lambda t: (0, 0)),
            pl.BlockSpec((1, M), lambda t: (0, 0)),
            pl.BlockSpec((M, D), lambda t: (0, 0)),
        ],
        out_specs=[
            pl.BlockSpec((B, TB, D), lambda t: (0, t, 0)),
            pl.BlockSpec((B, TB, D), lambda t: (0, t, 0)),
            pl.BlockSpec((1, 1), lambda t: (0, 0)),
            pl.BlockSpec((1, 1), lambda t: (0, 0)),
            pl.BlockSpec((M, D), lambda t: (0, 0)),
        ],
        out_shape=[
            jax.ShapeDtypeStruct((B, T, D), jnp.float32),
            jax.ShapeDtypeStruct((B, T, D), jnp.float32),
            jax.ShapeDtypeStruct((1, 1), jnp.float32),
            jax.ShapeDtypeStruct((1, 1), jnp.int32),
            jax.ShapeDtypeStruct((M, D), jnp.float32),
        ],
        scratch_shapes=[
            pltpu.VMEM((T, B, B), jnp.float32),
            pltpu.VMEM((T, B, B), jnp.float32),
            pltpu.VMEM((M, D), jnp.float32),
            pltpu.VMEM((M, D), jnp.float32),
            pltpu.VMEM((1, M), jnp.float32),
            pltpu.VMEM((1, M), jnp.float32),
            pltpu.VMEM((B, M), jnp.float32),
            pltpu.VMEM((B, M), jnp.float32),
        ],
    )(audio_semantic, video_semantic, embedding,
      ema_count.reshape(1, M), ema_weight)

    return (a_q, v_q, cm[0, 0], eq[0, 0], new_embedding)


# speedup vs baseline: 4.6183x; 1.1127x over previous
import jax
import jax.numpy as jnp
import numpy as np
from jax.experimental import pallas as pl
from jax.experimental.pallas import tpu as pltpu

B, T, D, M = 128, 64, 256, 1024
TB = 8
NT = TB * B
NBLK = T // TB
DECAY, EPS = 0.99, 1e-05
MAXENT = np.log(M)


def _lcmcm_from_scode(S):
    mx = jnp.max(-S)
    es = jnp.exp(S + mx)
    sums = jnp.sum(es, axis=-1)
    eye = (jax.lax.broadcasted_iota(jnp.int32, (B, B), 0)
           == jax.lax.broadcasted_iota(jnp.int32, (B, B), 1)).astype(jnp.float32)
    diag_s = jnp.sum(S * eye[None, :, :], axis=-1)
    return -jnp.mean(diag_s + mx - jnp.log(sums + EPS))


def _fused_kernel(a_ref, v_ref, emb_ref, cnt_ref, wgt_ref,
                  aq_ref, vq_ref, cm_ref, eq_ref, emb_out_ref,
                  s1_ref, s2_ref, wsa_ref, wsv_ref,
                  csa_ref, csv_ref, ca_ref, cv_ref):
    i = pl.program_id(0)
    emb = emb_ref[...]
    e2 = jnp.sum(emb * emb, axis=1, keepdims=True).T

    def encode(x, y):
        x2 = jnp.sum(x * x, axis=1, keepdims=True)
        xe = jax.lax.dot_general(x, emb, (((1,), (1,)), ((), ())))
        d = e2 + x2 - 2.0 * xe

        iota_m = jax.lax.broadcasted_iota(jnp.int32, (NT, M), 1)
        dmin = jnp.min(d, axis=1, keepdims=True)
        onehot = (jnp.min(jnp.where(d == dmin, iota_m, M), axis=1, keepdims=True)
                  == iota_m).astype(jnp.float32)

        ez1 = jnp.exp(jnp.sqrt(jnp.maximum(dmin, 0.0)) - jnp.sqrt(jnp.maximum(d, 0.0)))
        s = jnp.sum(ez1, axis=1, keepdims=True)
        logs = jnp.log(s)
        ez1sq = ez1 * ez1
        s2 = jnp.sum(ez1sq, axis=1, keepdims=True)

        ent = logs - jnp.sum(ez1 * jnp.log(ez1 + 1e-5 * s), axis=1, keepdims=True) / s
        adj = 1.0 - ent / MAXENT

        llog10 = jnp.log(ez1 + 1e-10 * s)

        q = jax.lax.dot_general(onehot, emb, (((1,), (0,)), ((), ())))
        w = jax.lax.dot_general(onehot, adj * (x + y),
                                (((0,), (0,)), ((), ())))
        c = jax.lax.dot_general(adj, onehot, (((0,), (0,)), ((), ())))
        return q, ez1sq, s2, logs, llog10, onehot, w, c

    x = jnp.swapaxes(a_ref[...], 0, 1).reshape(NT, D)
    y = jnp.swapaxes(v_ref[...], 0, 1).reshape(NT, D)
    qa, num05a, den05a, logsa, llog10a, oha, wa, csa = encode(x, y)
    qv, num05v, den05v, logsv, llog10v, ohv, wv, csv = encode(y, x)
    aq_ref[...] = jnp.swapaxes(qa.reshape(TB, B, D), 0, 1)
    vq_ref[...] = jnp.swapaxes(qv.reshape(TB, B, D), 0, 1)

    s1 = []
    s2 = []
    for tt in range(TB):
        sl = slice(tt * B, (tt + 1) * B)
        su1 = jax.lax.dot_general(
            num05a[sl], llog10v[sl], (((1,), (1,)), ((), ())))
        s1.append(su1 / den05a[sl] - logsv[sl].T)
        su2 = jax.lax.dot_general(
            num05v[sl], llog10a[sl], (((1,), (1,)), ((), ())))
        s2.append(su2 / den05v[sl] - logsa[sl].T)
    s1_ref[pl.ds(i * TB, TB)] = jnp.stack(s1, axis=0)
    s2_ref[pl.ds(i * TB, TB)] = jnp.stack(s2, axis=0)

    cnt_a = jnp.sum(oha.reshape(TB, B, M), axis=0)
    cnt_v = jnp.sum(ohv.reshape(TB, B, M), axis=0)

    @pl.when(i == 0)
    def _init():
        wsa_ref[...] = wa
        wsv_ref[...] = wv
        csa_ref[...] = csa
        csv_ref[...] = csv
        ca_ref[...] = cnt_a
        cv_ref[...] = cnt_v

    @pl.when(i > 0)
    def _acc():
        wsa_ref[...] += wa
        wsv_ref[...] += wv
        csa_ref[...] += csa
        csv_ref[...] += csv
        ca_ref[...] += cnt_a
        cv_ref[...] += cnt_v

    @pl.when(i == NBLK - 1)
    def _finale():
        l1 = _lcmcm_from_scode(s1_ref[...])
        l2 = _lcmcm_from_scode(s2_ref[...])
        cm_ref[...] = jnp.reshape(0.5 * (l1 + l2), (1, 1))

        counts_a = ca_ref[...]
        counts_v = cv_ref[...]
        iota_b = jax.lax.broadcasted_iota(jnp.int32, (B, M), 1)
        ama = jnp.min(jnp.where(counts_a == jnp.max(counts_a, axis=1, keepdims=True),
                                iota_b, M), axis=1, keepdims=True)
        amv = jnp.min(jnp.where(counts_v == jnp.max(counts_v, axis=1, keepdims=True),
                                iota_b, M), axis=1, keepdims=True)
        eq_ref[...] = jnp.reshape(jnp.sum((ama == amv).astype(jnp.int32)), (1, 1))

        ec = DECAY * cnt_ref[...] + (1.0 - DECAY) * csv_ref[...]
        n = jnp.sum(ec)
        ec = (ec + EPS) / (n + M * EPS) * n
        ew = DECAY * wgt_ref[...] + 0.5 * (1.0 - DECAY) * wsv_ref[...]
        ec2 = DECAY * ec + (1.0 - DECAY) * csa_ref[...]
        n2 = jnp.sum(ec2)
        ec2 = (ec2 + EPS) / (n2 + M * EPS) * n2
        ew2 = DECAY * ew + 0.5 * (1.0 - DECAY) * wsa_ref[...]
        emb_out_ref[...] = ew2 / ec2.T


def kernel(audio_semantic, video_semantic, embedding, ema_count, ema_weight):
    a_q, v_q, cm, eq, new_embedding = pl.pallas_call(
        _fused_kernel,
        grid=(NBLK,),
        in_specs=[
            pl.BlockSpec((B, TB, D), lambda t: (0, t, 0)),
            pl.BlockSpec((B, TB, D), lambda t: (0, t, 0)),
            pl.BlockSpec((M, D), lambda t: (0, 0)),
            pl.BlockSpec((1, M), lambda t: (0, 0)),
            pl.BlockSpec((M, D), lambda t: (0, 0)),
        ],
        out_specs=[
            pl.BlockSpec((B, TB, D), lambda t: (0, t, 0)),
            pl.BlockSpec((B, TB, D), lambda t: (0, t, 0)),
            pl.BlockSpec((1, 1), lambda t: (0, 0)),
            pl.BlockSpec((1, 1), lambda t: (0, 0)),
            pl.BlockSpec((M, D), lambda t: (0, 0)),
        ],
        out_shape=[
            jax.ShapeDtypeStruct((B, T, D), jnp.float32),
            jax.ShapeDtypeStruct((B, T, D), jnp.float32),
            jax.ShapeDtypeStruct((1, 1), jnp.float32),
            jax.ShapeDtypeStruct((1, 1), jnp.int32),
            jax.ShapeDtypeStruct((M, D), jnp.float32),
        ],
        scratch_shapes=[
            pltpu.VMEM((T, B, B), jnp.float32),
            pltpu.VMEM((T, B, B), jnp.float32),
            pltpu.VMEM((M, D), jnp.float32),
            pltpu.VMEM((M, D), jnp.float32),
            pltpu.VMEM((1, M), jnp.float32),
            pltpu.VMEM((1, M), jnp.float32),
            pltpu.VMEM((B, M), jnp.float32),
            pltpu.VMEM((B, M), jnp.float32),
        ],
    )(audio_semantic, video_semantic, embedding,
      ema_count.reshape(1, M), ema_weight)

    return (a_q, v_q, cm[0, 0], eq[0, 0], new_embedding)


# speedup vs baseline: 4.7094x; 1.0197x over previous
import jax
import jax.numpy as jnp
import numpy as np
from jax.experimental import pallas as pl
from jax.experimental.pallas import tpu as pltpu

B, T, D, M = 128, 64, 256, 1024
TB = 8
NT = TB * B
NBLK = T // TB
DECAY, EPS = 0.99, 1e-05
MAXENT = np.log(M)


def _lcmcm_from_scode(S):
    mx = jnp.max(-S)
    es = jnp.exp(S + mx)
    sums = jnp.sum(es, axis=-1)
    eye = (jax.lax.broadcasted_iota(jnp.int32, (B, B), 0)
           == jax.lax.broadcasted_iota(jnp.int32, (B, B), 1)).astype(jnp.float32)
    diag_s = jnp.sum(S * eye[None, :, :], axis=-1)
    return -jnp.mean(diag_s + mx - jnp.log(sums + EPS))


def _fused_kernel(a_ref, v_ref, emb_ref, cnt_ref, wgt_ref,
                  aq_ref, vq_ref, cm_ref, eq_ref, emb_out_ref,
                  s1_ref, s2_ref, wsa_ref, wsv_ref,
                  csa_ref, csv_ref, ca_ref, cv_ref,
                  emb2_ref, e2_ref):
    i = pl.program_id(0)
    emb = emb_ref[...]

    @pl.when(i == 0)
    def _precomp():
        emb2_ref[...] = emb + emb
        e2_ref[...] = jnp.sum(emb * emb, axis=1, keepdims=True).T

    emb2 = emb2_ref[...]
    e2 = e2_ref[...]

    def encode(x, y):
        x2 = jnp.sum(x * x, axis=1, keepdims=True)
        xe2 = jax.lax.dot_general(x, emb2, (((1,), (1,)), ((), ())))
        d = (e2 + x2) - xe2

        iota_m = jax.lax.broadcasted_iota(jnp.int32, (NT, M), 1)
        dmin = jnp.min(d, axis=1, keepdims=True)
        onehot = (jnp.min(jnp.where(d == dmin, iota_m, M), axis=1, keepdims=True)
                  == iota_m).astype(jnp.float32)

        ez1 = jnp.exp(jnp.sqrt(jnp.maximum(dmin, 0.0)) - jnp.sqrt(jnp.maximum(d, 0.0)))
        s = jnp.sum(ez1, axis=1, keepdims=True)
        logs = jnp.log(s)
        ez1sq = ez1 * ez1
        s2 = jnp.sum(ez1sq, axis=1, keepdims=True)

        ent = logs - jnp.sum(ez1 * jnp.log(ez1 + 1e-5 * s), axis=1, keepdims=True) / s
        adj = 1.0 - ent / MAXENT

        llog10 = jnp.log(ez1 + 1e-10 * s)

        q = jax.lax.dot_general(onehot, emb, (((1,), (0,)), ((), ())))
        w = jax.lax.dot_general(onehot, adj * (x + y),
                                (((0,), (0,)), ((), ())))
        c = jax.lax.dot_general(adj, onehot, (((0,), (0,)), ((), ())))
        return q, ez1sq, s2, logs, llog10, onehot, w, c

    x = jnp.swapaxes(a_ref[...], 0, 1).reshape(NT, D)
    y = jnp.swapaxes(v_ref[...], 0, 1).reshape(NT, D)
    qa, num05a, den05a, logsa, llog10a, oha, wa, csa = encode(x, y)
    qv, num05v, den05v, logsv, llog10v, ohv, wv, csv = encode(y, x)
    aq_ref[...] = jnp.swapaxes(qa.reshape(TB, B, D), 0, 1)
    vq_ref[...] = jnp.swapaxes(qv.reshape(TB, B, D), 0, 1)

    s1 = []
    s2 = []
    for tt in range(TB):
        sl = slice(tt * B, (tt + 1) * B)
        su1 = jax.lax.dot_general(
            num05a[sl], llog10v[sl], (((1,), (1,)), ((), ())))
        s1.append(su1 / den05a[sl] - logsv[sl].T)
        su2 = jax.lax.dot_general(
            num05v[sl], llog10a[sl], (((1,), (1,)), ((), ())))
        s2.append(su2 / den05v[sl] - logsa[sl].T)
    s1_ref[pl.ds(i * TB, TB)] = jnp.stack(s1, axis=0)
    s2_ref[pl.ds(i * TB, TB)] = jnp.stack(s2, axis=0)

    cnt_a = jnp.sum(oha.reshape(TB, B, M), axis=0)
    cnt_v = jnp.sum(ohv.reshape(TB, B, M), axis=0)

    @pl.when(i == 0)
    def _init():
        wsa_ref[...] = wa
        wsv_ref[...] = wv
        csa_ref[...] = csa
        csv_ref[...] = csv
        ca_ref[...] = cnt_a
        cv_ref[...] = cnt_v

    @pl.when(i > 0)
    def _acc():
        wsa_ref[...] += wa
        wsv_ref[...] += wv
        csa_ref[...] += csa
        csv_ref[...] += csv
        ca_ref[...] += cnt_a
        cv_ref[...] += cnt_v

    @pl.when(i == NBLK - 1)
    def _finale():
        l1 = _lcmcm_from_scode(s1_ref[...])
        l2 = _lcmcm_from_scode(s2_ref[...])
        cm_ref[...] = jnp.reshape(0.5 * (l1 + l2), (1, 1))

        counts_a = ca_ref[...]
        counts_v = cv_ref[...]
        iota_b = jax.lax.broadcasted_iota(jnp.int32, (B, M), 1)
        ama = jnp.min(jnp.where(counts_a == jnp.max(counts_a, axis=1, keepdims=True),
                                iota_b, M), axis=1, keepdims=True)
        amv = jnp.min(jnp.where(counts_v == jnp.max(counts_v, axis=1, keepdims=True),
                                iota_b, M), axis=1, keepdims=True)
        eq_ref[...] = jnp.reshape(jnp.sum((ama == amv).astype(jnp.int32)), (1, 1))

        ec = DECAY * cnt_ref[...] + (1.0 - DECAY) * csv_ref[...]
        n = jnp.sum(ec)
        ec = (ec + EPS) / (n + M * EPS) * n
        ew = DECAY * wgt_ref[...] + 0.5 * (1.0 - DECAY) * wsv_ref[...]
        ec2 = DECAY * ec + (1.0 - DECAY) * csa_ref[...]
        n2 = jnp.sum(ec2)
        ec2 = (ec2 + EPS) / (n2 + M * EPS) * n2
        ew2 = DECAY * ew + 0.5 * (1.0 - DECAY) * wsa_ref[...]
        emb_out_ref[...] = ew2 / ec2.T


def kernel(audio_semantic, video_semantic, embedding, ema_count, ema_weight):
    a_q, v_q, cm, eq, new_embedding = pl.pallas_call(
        _fused_kernel,
        grid=(NBLK,),
        in_specs=[
            pl.BlockSpec((B, TB, D), lambda t: (0, t, 0)),
            pl.BlockSpec((B, TB, D), lambda t: (0, t, 0)),
            pl.BlockSpec((M, D), lambda t: (0, 0)),
            pl.BlockSpec((1, M), lambda t: (0, 0)),
            pl.BlockSpec((M, D), lambda t: (0, 0)),
        ],
        out_specs=[
            pl.BlockSpec((B, TB, D), lambda t: (0, t, 0)),
            pl.BlockSpec((B, TB, D), lambda t: (0, t, 0)),
            pl.BlockSpec((1, 1), lambda t: (0, 0)),
            pl.BlockSpec((1, 1), lambda t: (0, 0)),
            pl.BlockSpec((M, D), lambda t: (0, 0)),
        ],
        out_shape=[
            jax.ShapeDtypeStruct((B, T, D), jnp.float32),
            jax.ShapeDtypeStruct((B, T, D), jnp.float32),
            jax.ShapeDtypeStruct((1, 1), jnp.float32),
            jax.ShapeDtypeStruct((1, 1), jnp.int32),
            jax.ShapeDtypeStruct((M, D), jnp.float32),
        ],
        scratch_shapes=[
            pltpu.VMEM((T, B, B), jnp.float32),
            pltpu.VMEM((T, B, B), jnp.float32),
            pltpu.VMEM((M, D), jnp.float32),
            pltpu.VMEM((M, D), jnp.float32),
            pltpu.VMEM((1, M), jnp.float32),
            pltpu.VMEM((1, M), jnp.float32),
            pltpu.VMEM((B, M), jnp.float32),
            pltpu.VMEM((B, M), jnp.float32),
            pltpu.VMEM((M, D), jnp.float32),
            pltpu.VMEM((1, M), jnp.float32),
        ],
    )(audio_semantic, video_semantic, embedding,
      ema_count.reshape(1, M), ema_weight)

    return (a_q, v_q, cm[0, 0], eq[0, 0], new_embedding)
